# baseline (speedup 1.0000x reference)
import functools

import jax
import jax.numpy as jnp
from jax import lax
from jax.experimental import pallas as pl
from jax.experimental.pallas import tpu as pltpu
from jax.experimental.pallas import tpu_sc as plsc

_HIDDEN = 128
_EPS = 1e-6
_NC = 2
_NS = 16
_NW = _NC * _NS
_LANES = 16
_HREG = _HIDDEN // _LANES
_CHUNK = 128


def _rsqrt_newton(x):
    i = lax.bitcast_convert_type(x, jnp.int32)
    magic = jnp.full((_LANES,), 0x5F3759DF, dtype=jnp.int32)
    one = jnp.full((_LANES,), 1, dtype=jnp.int32)
    y = lax.bitcast_convert_type(magic - lax.shift_right_logical(i, one),
                                 jnp.float32)
    half_x = 0.5 * x
    for _ in range(3):
        y = y * (1.5 - half_x * y * y)
    return y


def _layernorm_chunk(rows_v, gam, bet):
    def row_body(r, carry):
        xs = [rows_v[r, pl.ds(_LANES * h, _LANES)] for h in range(_HREG)]
        acc = xs[0]
        acc2 = xs[0] * xs[0]
        for h in range(1, _HREG):
            acc = acc + xs[h]
            acc2 = acc2 + xs[h] * xs[h]
        s1 = jnp.sum(acc)
        s2 = jnp.sum(acc2)
        mean = s1 * (1.0 / _HIDDEN)
        var = s2 * (1.0 / _HIDDEN) - mean * mean
        rstd = _rsqrt_newton(jnp.full((_LANES,), var + _EPS, dtype=jnp.float32))
        mean_v = jnp.full((_LANES,), mean, dtype=jnp.float32)
        for h in range(_HREG):
            a = rstd * gam[h]
            rows_v[r, pl.ds(_LANES * h, _LANES)] = (xs[h] - mean_v) * a + bet[h]
        return carry

    lax.fori_loop(0, _CHUNK, row_body, 0)


def _make_kernel(n_tok):
    per_w = n_tok // _NW
    n_chunk = per_w // _CHUNK
    mesh = plsc.VectorSubcoreMesh(core_axis_name="c", subcore_axis_name="s")

    @functools.partial(
        pl.kernel,
        out_type=jax.ShapeDtypeStruct((n_tok, _HIDDEN), jnp.float32),
        mesh=mesh,
        compiler_params=pltpu.CompilerParams(needs_layout_passes=False),
        scratch_types=[
            pltpu.VMEM((n_chunk, _CHUNK), jnp.int32),
            pltpu.VMEM((_CHUNK, _HIDDEN), jnp.float32),
            pltpu.VMEM((_HIDDEN,), jnp.float32),
            pltpu.VMEM((_HIDDEN,), jnp.float32),
            pltpu.SemaphoreType.DMA,
        ],
    )
    def k(ids_hbm, table_hbm, gamma_hbm, beta_hbm, out_hbm,
          idx_v, rows_v, gv, bv, sem):
        c = lax.axis_index("c")
        s = lax.axis_index("s")
        wid = s * _NC + c
        pltpu.sync_copy(gamma_hbm, gv)
        pltpu.sync_copy(beta_hbm, bv)
        gam = [gv[pl.ds(_LANES * h, _LANES)] for h in range(_HREG)]
        bet = [bv[pl.ds(_LANES * h, _LANES)] for h in range(_HREG)]
        pltpu.sync_copy(ids_hbm.at[wid], idx_v)
        row0 = wid * per_w

        def chunk_body(j, carry):
            pltpu.async_copy(table_hbm.at[idx_v.at[j]], rows_v, sem).wait()
            _layernorm_chunk(rows_v, gam, bet)
            pltpu.sync_copy(rows_v,
                            out_hbm.at[pl.ds(row0 + j * _CHUNK, _CHUNK)])
            return carry

        lax.fori_loop(0, n_chunk, chunk_body, 0)

    return k


def kernel(input_ids, table, gamma, beta):
    b, seq = input_ids.shape
    n_tok = b * seq
    ids = input_ids.astype(jnp.int32).reshape(
        _NW, n_tok // (_NW * _CHUNK), _CHUNK)
    out = _make_kernel(n_tok)(ids, table, gamma, beta)
    return out.reshape(b, seq, _HIDDEN)


# speedup vs baseline: 3.4412x; 3.4412x over previous
import functools

import jax
import jax.numpy as jnp
from jax import lax
from jax.experimental import pallas as pl
from jax.experimental.pallas import tpu as pltpu
from jax.experimental.pallas import tpu_sc as plsc

_HIDDEN = 128
_EPS = 1e-6
_NC = 2
_NS = 16
_NW = _NC * _NS
_LANES = 16
_HREG = _HIDDEN // _LANES
_CHUNK = 128
_NB = 3


def _rsqrt_newton(x):
    i = lax.bitcast_convert_type(x, jnp.int32)
    magic = jnp.full((_LANES,), 0x5F3759DF, dtype=jnp.int32)
    one = jnp.full((_LANES,), 1, dtype=jnp.int32)
    y = lax.bitcast_convert_type(magic - lax.shift_right_logical(i, one),
                                 jnp.float32)
    half_x = 0.5 * x
    for _ in range(3):
        y = y * (1.5 - half_x * y * y)
    return y


def _layernorm_chunk(rows_v, b, gam, bet):
    @plsc.parallel_loop(0, _CHUNK, unroll=2)
    def _row(r):
        xs = [rows_v[b, r, pl.ds(_LANES * h, _LANES)] for h in range(_HREG)]
        acc = xs[0]
        acc2 = xs[0] * xs[0]
        for h in range(1, _HREG):
            acc = acc + xs[h]
            acc2 = acc2 + xs[h] * xs[h]
        s1 = jnp.sum(acc)
        s2 = jnp.sum(acc2)
        mean = s1 * (1.0 / _HIDDEN)
        var = s2 * (1.0 / _HIDDEN) - mean * mean
        rstd = _rsqrt_newton(jnp.full((_LANES,), var + _EPS,
                                      dtype=jnp.float32))
        mean_v = jnp.full((_LANES,), mean, dtype=jnp.float32)
        for h in range(_HREG):
            a = rstd * gam[h]
            rows_v[b, r, pl.ds(_LANES * h, _LANES)] = \
                (xs[h] - mean_v) * a + bet[h]


def _make_kernel(n_tok):
    per_w = n_tok // _NW
    n_chunk = per_w // _CHUNK
    mesh = plsc.VectorSubcoreMesh(core_axis_name="c", subcore_axis_name="s")

    @functools.partial(
        pl.kernel,
        out_type=jax.ShapeDtypeStruct((n_tok, _HIDDEN), jnp.float32),
        mesh=mesh,
        compiler_params=pltpu.CompilerParams(needs_layout_passes=False),
        scratch_types=[
            pltpu.VMEM((n_chunk, _CHUNK), jnp.int32),
            pltpu.VMEM((_NB, _CHUNK, _HIDDEN), jnp.float32),
            pltpu.VMEM((_HIDDEN,), jnp.float32),
            pltpu.VMEM((_HIDDEN,), jnp.float32),
            pltpu.SemaphoreType.DMA((_NB,)),
            pltpu.SemaphoreType.DMA((_NB,)),
        ],
    )
    def k(ids_hbm, table_hbm, gamma_hbm, beta_hbm, out_hbm,
          idx_v, rows_v, gv, bv, semg, semo):
        c = lax.axis_index("c")
        s = lax.axis_index("s")
        wid = s * _NC + c
        pltpu.sync_copy(gamma_hbm, gv)
        pltpu.sync_copy(beta_hbm, bv)
        gam = [gv[pl.ds(_LANES * h, _LANES)] for h in range(_HREG)]
        bet = [bv[pl.ds(_LANES * h, _LANES)] for h in range(_HREG)]
        pltpu.sync_copy(ids_hbm.at[wid], idx_v)
        row0 = wid * per_w

        def out_slice(g):
            return out_hbm.at[pl.ds(row0 + g * _CHUNK, _CHUNK)]

        pltpu.async_copy(table_hbm.at[idx_v.at[0]], rows_v.at[0], semg.at[0])

        def chunk_body(g, carry):
            b = lax.rem(g, _NB)
            nb = lax.rem(g + 1, _NB)

            @pl.when(g >= _NB - 1)
            def _():
                pltpu.make_async_copy(rows_v.at[nb], out_slice(g - 2),
                                      semo.at[nb]).wait()

            @pl.when(g + 1 < n_chunk)
            def _():
                pltpu.async_copy(table_hbm.at[idx_v.at[g + 1]],
                                 rows_v.at[nb], semg.at[nb])

            pltpu.make_async_copy(table_hbm.at[idx_v.at[b]], rows_v.at[b],
                                  semg.at[b]).wait()
            _layernorm_chunk(rows_v, b, gam, bet)
            pltpu.async_copy(rows_v.at[b], out_slice(g), semo.at[b])
            return carry

        lax.fori_loop(0, n_chunk, chunk_body, 0)

        for g in (n_chunk - 2, n_chunk - 1):
            pltpu.make_async_copy(rows_v.at[g % _NB], out_slice(g),
                                  semo.at[g % _NB]).wait()

    return k


def kernel(input_ids, table, gamma, beta):
    b, seq = input_ids.shape
    n_tok = b * seq
    ids = input_ids.astype(jnp.int32).reshape(
        _NW, n_tok // (_NW * _CHUNK), _CHUNK)
    out = _make_kernel(n_tok)(ids, table, gamma, beta)
    return out.reshape(b, seq, _HIDDEN)
